# initial kernel scaffold (unmeasured)
import jax
import jax.numpy as jnp
from jax import lax
from jax.experimental import pallas as pl
from jax.experimental.pallas import tpu as pltpu

N_DEV = 8
B = 2
SQ = 512
SKV = 4096
SKV_LOC = 512
H_LOC = 8
DH = 64
E = 768
BLK = 64


def kernel(x, Wq, K_ext, V_ext, Wo):
    bf16 = jnp.bfloat16
    f32 = jnp.float32

    def body(x_ref, wq_ref, k_ref, v_ref, wo_ref, out_ref,
             ksend, vsend, kg, vg, qbuf, ctx_buf, ar_buf,
             k_send_sems, k_recv_sems, v_send_sems, v_recv_sems,
             ar_send_sems, ar_recv_sems):
        me = lax.axis_index("i")

        bsem = pltpu.get_barrier_semaphore()
        for d in range(1, N_DEV):
            pl.semaphore_signal(
                bsem, inc=1,
                device_id=((me + d) % N_DEV,),
                device_id_type=pl.DeviceIdType.MESH,
            )
        pl.semaphore_wait(bsem, N_DEV - 1)

        for p in range(N_DEV):
            ksend[p] = k_ref[:, :, pl.ds(p * H_LOC, H_LOC), :].astype(bf16)
            vsend[p] = v_ref[:, :, pl.ds(p * H_LOC, H_LOC), :].astype(bf16)
        kg[me] = ksend[me]
        vg[me] = vsend[me]

        dmas = []
        for h in range(1, N_DEV):
            dst = (me + h) % N_DEV
            for src_buf, dst_buf, ssems, rsems in (
                (ksend, kg, k_send_sems, k_recv_sems),
                (vsend, vg, v_send_sems, v_recv_sems),
            ):
                dma = pltpu.make_async_remote_copy(
                    src_ref=src_buf.at[dst],
                    dst_ref=dst_buf.at[me],
                    send_sem=ssems.at[h - 1],
                    recv_sem=rsems.at[h - 1],
                    device_id=(dst,),
                    device_id_type=pl.DeviceIdType.MESH,
                )
                dma.start()
                dmas.append(dma)

        wq = wq_ref[...].astype(bf16)
        for b in range(B):
            q = jnp.dot(x_ref[b].astype(bf16), wq,
                        preferred_element_type=f32) * 0.125
            qbuf[b] = q.astype(bf16).reshape(SQ, H_LOC, DH)

        for dma in dmas:
            dma.wait()

        rows = lax.broadcasted_iota(jnp.int32, (SQ, SKV), 0)
        cols = lax.broadcasted_iota(jnp.int32, (SQ, SKV), 1)
        keep = ((rows // BLK) % 4) == ((cols // BLK) % 4)

        def att_body(i, carry):
            b = i // H_LOC
            h = i % H_LOC
            q = qbuf[b, :, h, :]
            kall = jnp.concatenate(
                [kg[s, b, :, h, :] for s in range(N_DEV)], axis=0)
            vall = jnp.concatenate(
                [vg[s, b, :, h, :] for s in range(N_DEV)], axis=0)
            s_ = lax.dot_general(q, kall, (((1,), (1,)), ((), ())),
                                 preferred_element_type=f32)
            s_ = jnp.where(keep, s_, -1e9)
            mx = jnp.max(s_, axis=-1, keepdims=True)
            w = jnp.exp(s_ - mx)
            den = jnp.sum(w, axis=-1, keepdims=True)
            wn = (w / den).astype(bf16)
            ctx = jnp.dot(wn, vall, preferred_element_type=f32)
            ctx_buf[b, :, pl.ds(h * DH, DH)] = ctx
            return carry

        lax.fori_loop(0, B * H_LOC, att_body, 0)

        wo = wo_ref[...].astype(bf16)
        for b in range(B):
            pb = jnp.dot(ctx_buf[b].astype(bf16), wo,
                         preferred_element_type=f32)
            out_ref[b] = pb
            ar_buf[0, b] = pb.astype(bf16)

        right = (me + 1) % N_DEV
        for h in range(1, N_DEV):
            dma = pltpu.make_async_remote_copy(
                src_ref=ar_buf.at[h - 1],
                dst_ref=ar_buf.at[h],
                send_sem=ar_send_sems.at[h - 1],
                recv_sem=ar_recv_sems.at[h - 1],
                device_id=(right,),
                device_id_type=pl.DeviceIdType.MESH,
            )
            dma.start()
            dma.wait()
            out_ref[...] = out_ref[...] + ar_buf[h].astype(f32)

    return pl.pallas_call(
        body,
        out_shape=jax.ShapeDtypeStruct((B, SQ, E), jnp.float32),
        in_specs=[pl.BlockSpec(memory_space=pltpu.VMEM)] * 5,
        out_specs=pl.BlockSpec(memory_space=pltpu.VMEM),
        scratch_shapes=[
            pltpu.VMEM((N_DEV, B, SKV_LOC, H_LOC, DH), bf16),
            pltpu.VMEM((N_DEV, B, SKV_LOC, H_LOC, DH), bf16),
            pltpu.VMEM((N_DEV, B, SKV_LOC, H_LOC, DH), bf16),
            pltpu.VMEM((N_DEV, B, SKV_LOC, H_LOC, DH), bf16),
            pltpu.VMEM((B, SQ, H_LOC, DH), bf16),
            pltpu.VMEM((B, SQ, H_LOC * DH), f32),
            pltpu.VMEM((N_DEV, B, SQ, E), bf16),
            pltpu.SemaphoreType.DMA((N_DEV - 1,)),
            pltpu.SemaphoreType.DMA((N_DEV - 1,)),
            pltpu.SemaphoreType.DMA((N_DEV - 1,)),
            pltpu.SemaphoreType.DMA((N_DEV - 1,)),
            pltpu.SemaphoreType.DMA((N_DEV - 1,)),
            pltpu.SemaphoreType.DMA((N_DEV - 1,)),
        ],
        compiler_params=pltpu.CompilerParams(collective_id=0),
    )(x, Wq, K_ext, V_ext, Wo)


# baseline (device time: 616181 ns/iter reference)
import jax
import jax.numpy as jnp
from jax import lax
from jax.experimental import pallas as pl
from jax.experimental.pallas import tpu as pltpu

N_DEV = 8
B = 2
SQ = 512
SKV = 4096
SKV_LOC = 512
H_LOC = 8
DH = 64
E = 768
BLK = 64
NCLS = 4
KROWS = SKV // NCLS


def kernel(x, Wq, K_ext, V_ext, Wo):
    bf16 = jnp.bfloat16
    f32 = jnp.float32

    perm = [0, 4, 1, 5, 2, 6, 3, 7]

    def prep(t):
        t = jnp.transpose(t.astype(bf16), (0, 2, 1, 3))
        t = t.reshape(B, 64, SKV_LOC // BLK, BLK, DH)[:, :, perm]
        return t.reshape(B, 64, NCLS, 2 * BLK, DH)

    Kt = prep(K_ext)
    Vt = prep(V_ext)

    def body(x_ref, wq_ref, kt_ref, vt_ref, wo_ref, out_ref,
             kg, vg, qbuf, ar_buf,
             k_send_sems, k_recv_sems, v_send_sems, v_recv_sems,
             ar_send_sems, ar_recv_sems, local_sems):
        me = lax.axis_index("i")

        bsem = pltpu.get_barrier_semaphore()
        for d in range(1, N_DEV):
            pl.semaphore_signal(
                bsem, inc=1,
                device_id=((me + d) % N_DEV,),
                device_id_type=pl.DeviceIdType.MESH,
            )
        pl.semaphore_wait(bsem, N_DEV - 1)

        dmas = []
        for h in range(1, N_DEV):
            dst = (me + h) % N_DEV
            for src_ref, dst_buf, ssems, rsems in (
                (kt_ref, kg, k_send_sems, k_recv_sems),
                (vt_ref, vg, v_send_sems, v_recv_sems),
            ):
                dma = pltpu.make_async_remote_copy(
                    src_ref=src_ref.at[:, pl.ds(dst * H_LOC, H_LOC)],
                    dst_ref=dst_buf.at[
                        :, :, :, pl.ds(me * 2 * BLK, 2 * BLK), :],
                    send_sem=ssems.at[h - 1],
                    recv_sem=rsems.at[h - 1],
                    device_id=(dst,),
                    device_id_type=pl.DeviceIdType.MESH,
                )
                dma.start()
                dmas.append(dma)

        local_dmas = []
        for idx, (src_ref, dst_buf) in enumerate(((kt_ref, kg), (vt_ref, vg))):
            ldma = pltpu.make_async_copy(
                src_ref.at[:, pl.ds(me * H_LOC, H_LOC)],
                dst_buf.at[:, :, :, pl.ds(me * 2 * BLK, 2 * BLK), :],
                local_sems.at[idx],
            )
            ldma.start()
            local_dmas.append(ldma)

        wq = wq_ref[...].astype(bf16)
        for b in range(B):
            q = jnp.dot(x_ref[b].astype(bf16), wq,
                        preferred_element_type=f32) * 0.125
            qs = q.astype(bf16)
            for h in range(H_LOC):
                qbuf[b * H_LOC + h] = qs[:, h * DH:(h + 1) * DH]

        out_ref[...] = jnp.zeros((B, SQ, E), f32)

        for ldma in local_dmas:
            ldma.wait()
        for dma in dmas:
            dma.wait()

        def att_body(i, carry):
            b = i // H_LOC
            h = i % H_LOC
            q_full = qbuf[i]
            wo_h = wo_ref[pl.ds(h * DH, DH), :].astype(bf16)
            for c in range(NCLS):
                q_c = jnp.concatenate(
                    [q_full[c * BLK:(c + 1) * BLK],
                     q_full[(c + NCLS) * BLK:(c + NCLS + 1) * BLK]],
                    axis=0)
                k_c = kg[b, h, c]
                v_c = vg[b, h, c]
                s_ = lax.dot_general(q_c, k_c, (((1,), (1,)), ((), ())),
                                     preferred_element_type=f32)
                mx = jnp.max(s_, axis=-1, keepdims=True)
                w = jnp.exp(s_ - mx)
                den = jnp.sum(w, axis=-1, keepdims=True)
                wn = (w / den).astype(bf16)
                ctx = jnp.dot(wn, v_c, preferred_element_type=f32)
                po = jnp.dot(ctx.astype(bf16), wo_h,
                             preferred_element_type=f32)
                r0 = pl.ds(c * BLK, BLK)
                r1 = pl.ds((c + NCLS) * BLK, BLK)
                out_ref[b, r0, :] = out_ref[b, r0, :] + po[0:BLK]
                out_ref[b, r1, :] = out_ref[b, r1, :] + po[BLK:2 * BLK]
            return carry

        lax.fori_loop(0, B * H_LOC, att_body, 0)

        ar_buf[0] = out_ref[...].astype(bf16)
        right = (me + 1) % N_DEV
        for h in range(1, N_DEV):
            dma = pltpu.make_async_remote_copy(
                src_ref=ar_buf.at[h - 1],
                dst_ref=ar_buf.at[h],
                send_sem=ar_send_sems.at[h - 1],
                recv_sem=ar_recv_sems.at[h - 1],
                device_id=(right,),
                device_id_type=pl.DeviceIdType.MESH,
            )
            dma.start()
            dma.wait()
            out_ref[...] = out_ref[...] + ar_buf[h].astype(f32)

    return pl.pallas_call(
        body,
        out_shape=jax.ShapeDtypeStruct((B, SQ, E), jnp.float32),
        in_specs=[
            pl.BlockSpec(memory_space=pltpu.VMEM),
            pl.BlockSpec(memory_space=pltpu.VMEM),
            pl.BlockSpec(memory_space=pltpu.MemorySpace.HBM),
            pl.BlockSpec(memory_space=pltpu.MemorySpace.HBM),
            pl.BlockSpec(memory_space=pltpu.VMEM),
        ],
        out_specs=pl.BlockSpec(memory_space=pltpu.VMEM),
        scratch_shapes=[
            pltpu.VMEM((B, H_LOC, NCLS, KROWS, DH), bf16),
            pltpu.VMEM((B, H_LOC, NCLS, KROWS, DH), bf16),
            pltpu.VMEM((B * H_LOC, SQ, DH), bf16),
            pltpu.VMEM((N_DEV, B, SQ, E), bf16),
            pltpu.SemaphoreType.DMA((N_DEV - 1,)),
            pltpu.SemaphoreType.DMA((N_DEV - 1,)),
            pltpu.SemaphoreType.DMA((N_DEV - 1,)),
            pltpu.SemaphoreType.DMA((N_DEV - 1,)),
            pltpu.SemaphoreType.DMA((N_DEV - 1,)),
            pltpu.SemaphoreType.DMA((N_DEV - 1,)),
            pltpu.SemaphoreType.DMA((2,)),
        ],
        compiler_params=pltpu.CompilerParams(
            collective_id=0,
            vmem_limit_bytes=60 * 1024 * 1024,
        ),
    )(x, Wq, Kt, Vt, Wo)


# device time: 539928 ns/iter; 1.1412x vs baseline; 1.1412x over previous
import jax
import jax.numpy as jnp
from jax import lax
from jax.experimental import pallas as pl
from jax.experimental.pallas import tpu as pltpu

N_DEV = 8
B = 2
SQ = 512
SKV = 4096
SKV_LOC = 512
H_LOC = 8
DH = 64
E = 768
BLK = 64
NCLS = 4
KROWS = SKV // NCLS
CH = SQ // N_DEV


def kernel(x, Wq, K_ext, V_ext, Wo):
    bf16 = jnp.bfloat16
    f32 = jnp.float32

    perm = [0, 4, 1, 5, 2, 6, 3, 7]

    def prep(t):
        t = jnp.transpose(t.astype(bf16), (0, 2, 1, 3))
        t = t.reshape(B, 64, SKV_LOC // BLK, BLK, DH)[:, :, perm]
        return t.reshape(B, 64, NCLS, 2 * BLK, DH)

    Kt = prep(K_ext)
    Vt = prep(V_ext)

    def body(x_ref, wq_ref, kt_ref, vt_ref, wo_ref, out_ref,
             kg, vg, qbuf, rs_send, rs_recv, ag_seed, ag_recv,
             k_send_sems, k_recv_sems, v_send_sems, v_recv_sems,
             rs_send_sems, rs_recv_sems, ag_send_sems, ag_recv_sems,
             local_sems):
        me = lax.axis_index("i")

        bsem = pltpu.get_barrier_semaphore()
        for d in range(1, N_DEV):
            pl.semaphore_signal(
                bsem, inc=1,
                device_id=((me + d) % N_DEV,),
                device_id_type=pl.DeviceIdType.MESH,
            )
        pl.semaphore_wait(bsem, N_DEV - 1)

        dmas = []
        for h in range(1, N_DEV):
            dst = (me + h) % N_DEV
            for src_ref, dst_buf, ssems, rsems in (
                (kt_ref, kg, k_send_sems, k_recv_sems),
                (vt_ref, vg, v_send_sems, v_recv_sems),
            ):
                dma = pltpu.make_async_remote_copy(
                    src_ref=src_ref.at[:, pl.ds(dst * H_LOC, H_LOC)],
                    dst_ref=dst_buf.at[me],
                    send_sem=ssems.at[h - 1],
                    recv_sem=rsems.at[h - 1],
                    device_id=(dst,),
                    device_id_type=pl.DeviceIdType.MESH,
                )
                dma.start()
                dmas.append(dma)

        local_dmas = []
        for idx, (src_ref, dst_buf) in enumerate(((kt_ref, kg), (vt_ref, vg))):
            ldma = pltpu.make_async_copy(
                src_ref.at[:, pl.ds(me * H_LOC, H_LOC)],
                dst_buf.at[me],
                local_sems.at[idx],
            )
            ldma.start()
            local_dmas.append(ldma)

        wq = wq_ref[...].astype(bf16)
        for b in range(B):
            q = jnp.dot(x_ref[b].astype(bf16), wq,
                        preferred_element_type=f32) * 0.125
            qs = q.astype(bf16)
            for h in range(H_LOC):
                qbuf[b * H_LOC + h] = qs[:, h * DH:(h + 1) * DH]

        out_ref[...] = jnp.zeros((B, SQ, E), f32)

        for ldma in local_dmas:
            ldma.wait()
        for dma in dmas:
            dma.wait()

        def att_body(i, carry):
            b = i // H_LOC
            h = i % H_LOC
            q_full = qbuf[i]
            wo_h = wo_ref[pl.ds(h * DH, DH), :].astype(bf16)
            for c in range(NCLS):
                q_c = jnp.concatenate(
                    [q_full[c * BLK:(c + 1) * BLK],
                     q_full[(c + NCLS) * BLK:(c + NCLS + 1) * BLK]],
                    axis=0)
                k_c = jnp.concatenate(
                    [kg[s, b, h, c] for s in range(N_DEV)], axis=0)
                v_c = jnp.concatenate(
                    [vg[s, b, h, c] for s in range(N_DEV)], axis=0)
                s_ = lax.dot_general(q_c, k_c, (((1,), (1,)), ((), ())),
                                     preferred_element_type=f32)
                mx = jnp.max(s_, axis=-1, keepdims=True)
                w = jnp.exp(s_ - mx)
                den = jnp.sum(w, axis=-1, keepdims=True)
                wn = (w / den).astype(bf16)
                ctx = jnp.dot(wn, v_c, preferred_element_type=f32)
                po = jnp.dot(ctx.astype(bf16), wo_h,
                             preferred_element_type=f32)
                r0 = pl.ds(c * BLK, BLK)
                r1 = pl.ds((c + NCLS) * BLK, BLK)
                out_ref[b, r0, :] = out_ref[b, r0, :] + po[0:BLK]
                out_ref[b, r1, :] = out_ref[b, r1, :] + po[BLK:2 * BLK]
            return carry

        lax.fori_loop(0, B * H_LOC, att_body, 0)

        right = (me + 1) % N_DEV

        for s in range(N_DEV - 1):
            c_send = (me - s) % N_DEV
            rs_send[s] = out_ref[:, pl.ds(c_send * CH, CH), :].astype(bf16)
            dma = pltpu.make_async_remote_copy(
                src_ref=rs_send.at[s],
                dst_ref=rs_recv.at[s],
                send_sem=rs_send_sems.at[s],
                recv_sem=rs_recv_sems.at[s],
                device_id=(right,),
                device_id_type=pl.DeviceIdType.MESH,
            )
            dma.start()
            dma.wait()
            c_recv = (me - 1 - s) % N_DEV
            r = pl.ds(c_recv * CH, CH)
            out_ref[:, r, :] = out_ref[:, r, :] + rs_recv[s].astype(f32)

        ag_seed[...] = out_ref[:, pl.ds(((me + 1) % N_DEV) * CH, CH),
                               :].astype(bf16)
        for s in range(N_DEV - 1):
            src = ag_seed if s == 0 else ag_recv.at[s - 1]
            dma = pltpu.make_async_remote_copy(
                src_ref=src,
                dst_ref=ag_recv.at[s],
                send_sem=ag_send_sems.at[s],
                recv_sem=ag_recv_sems.at[s],
                device_id=(right,),
                device_id_type=pl.DeviceIdType.MESH,
            )
            dma.start()
            dma.wait()
            c = (me - s) % N_DEV
            out_ref[:, pl.ds(c * CH, CH), :] = ag_recv[s].astype(f32)

    return pl.pallas_call(
        body,
        out_shape=jax.ShapeDtypeStruct((B, SQ, E), jnp.float32),
        in_specs=[
            pl.BlockSpec(memory_space=pltpu.VMEM),
            pl.BlockSpec(memory_space=pltpu.VMEM),
            pl.BlockSpec(memory_space=pltpu.MemorySpace.HBM),
            pl.BlockSpec(memory_space=pltpu.MemorySpace.HBM),
            pl.BlockSpec(memory_space=pltpu.VMEM),
        ],
        out_specs=pl.BlockSpec(memory_space=pltpu.VMEM),
        scratch_shapes=[
            pltpu.VMEM((N_DEV, B, H_LOC, NCLS, 2 * BLK, DH), bf16),
            pltpu.VMEM((N_DEV, B, H_LOC, NCLS, 2 * BLK, DH), bf16),
            pltpu.VMEM((B * H_LOC, SQ, DH), bf16),
            pltpu.VMEM((N_DEV - 1, B, CH, E), bf16),
            pltpu.VMEM((N_DEV - 1, B, CH, E), bf16),
            pltpu.VMEM((B, CH, E), bf16),
            pltpu.VMEM((N_DEV - 1, B, CH, E), bf16),
            pltpu.SemaphoreType.DMA((N_DEV - 1,)),
            pltpu.SemaphoreType.DMA((N_DEV - 1,)),
            pltpu.SemaphoreType.DMA((N_DEV - 1,)),
            pltpu.SemaphoreType.DMA((N_DEV - 1,)),
            pltpu.SemaphoreType.DMA((N_DEV - 1,)),
            pltpu.SemaphoreType.DMA((N_DEV - 1,)),
            pltpu.SemaphoreType.DMA((N_DEV - 1,)),
            pltpu.SemaphoreType.DMA((N_DEV - 1,)),
            pltpu.SemaphoreType.DMA((2,)),
        ],
        compiler_params=pltpu.CompilerParams(
            collective_id=0,
            vmem_limit_bytes=60 * 1024 * 1024,
        ),
    )(x, Wq, Kt, Vt, Wo)


# device time: 467911 ns/iter; 1.3169x vs baseline; 1.1539x over previous
import jax
import jax.numpy as jnp
from jax import lax
from jax.experimental import pallas as pl
from jax.experimental.pallas import tpu as pltpu

N_DEV = 8
B = 2
SQ = 512
SKV = 4096
SKV_LOC = 512
H_LOC = 8
DH = 64
E = 768
BLK = 64
NCLS = 4
KROWS = SKV // NCLS
CH = SQ // N_DEV


def kernel(x, Wq, K_ext, V_ext, Wo):
    bf16 = jnp.bfloat16
    f32 = jnp.float32

    def prep(t):
        t = t.astype(bf16).reshape(B, 2, NCLS, BLK, 64, DH)
        t = jnp.transpose(t, (0, 4, 2, 1, 3, 5))
        return t.reshape(B, 64, NCLS, 2 * BLK, DH)

    Kt = prep(K_ext)
    Vt = prep(V_ext)

    def body(x_ref, wq_ref, kt_ref, vt_ref, wo_ref, out_ref,
             kg, vg, qbuf, rs_send, rs_recv, ag_seed, ag_recv,
             k_send_sems, k_recv_sems, v_send_sems, v_recv_sems,
             rs_send_sems, rs_recv_sems, ag_send_sems, ag_recv_sems,
             local_sems):
        me = lax.axis_index("i")

        bsem = pltpu.get_barrier_semaphore()
        for d in range(1, N_DEV):
            pl.semaphore_signal(
                bsem, inc=1,
                device_id=((me + d) % N_DEV,),
                device_id_type=pl.DeviceIdType.MESH,
            )
        pl.semaphore_wait(bsem, N_DEV - 1)

        dmas = []
        for h in range(1, N_DEV):
            dst = (me + h) % N_DEV
            for src_ref, dst_buf, ssems, rsems in (
                (kt_ref, kg, k_send_sems, k_recv_sems),
                (vt_ref, vg, v_send_sems, v_recv_sems),
            ):
                dma = pltpu.make_async_remote_copy(
                    src_ref=src_ref.at[:, pl.ds(dst * H_LOC, H_LOC)],
                    dst_ref=dst_buf.at[me],
                    send_sem=ssems.at[h - 1],
                    recv_sem=rsems.at[h - 1],
                    device_id=(dst,),
                    device_id_type=pl.DeviceIdType.MESH,
                )
                dma.start()
                dmas.append(dma)

        local_dmas = []
        for idx, (src_ref, dst_buf) in enumerate(((kt_ref, kg), (vt_ref, vg))):
            ldma = pltpu.make_async_copy(
                src_ref.at[:, pl.ds(me * H_LOC, H_LOC)],
                dst_buf.at[me],
                local_sems.at[idx],
            )
            ldma.start()
            local_dmas.append(ldma)

        wq = wq_ref[...].astype(bf16)
        for b in range(B):
            q = jnp.dot(x_ref[b].astype(bf16), wq,
                        preferred_element_type=f32) * 0.125
            qs = q.astype(bf16)
            for h in range(H_LOC):
                qbuf[b * H_LOC + h] = qs[:, h * DH:(h + 1) * DH]

        out_ref[...] = jnp.zeros((B, SQ, E), f32)

        for ldma in local_dmas:
            ldma.wait()
        for dma in dmas:
            dma.wait()

        def att_body(i, carry):
            b = i // H_LOC
            h = i % H_LOC
            q_full = qbuf[i]
            wo_h = wo_ref[pl.ds(h * DH, DH), :].astype(bf16)
            for c in range(NCLS):
                q_c = jnp.concatenate(
                    [q_full[c * BLK:(c + 1) * BLK],
                     q_full[(c + NCLS) * BLK:(c + NCLS + 1) * BLK]],
                    axis=0)
                k_c = jnp.concatenate(
                    [kg[s, b, h, c] for s in range(N_DEV)], axis=0)
                v_c = jnp.concatenate(
                    [vg[s, b, h, c] for s in range(N_DEV)], axis=0)
                s_ = lax.dot_general(q_c, k_c, (((1,), (1,)), ((), ())),
                                     preferred_element_type=f32)
                mx = jnp.max(s_, axis=-1, keepdims=True)
                w = jnp.exp(s_ - mx)
                den = jnp.sum(w, axis=-1, keepdims=True)
                wn = (w / den).astype(bf16)
                ctx = jnp.dot(wn, v_c, preferred_element_type=f32)
                po = jnp.dot(ctx.astype(bf16), wo_h,
                             preferred_element_type=f32)
                r0 = pl.ds(c * BLK, BLK)
                r1 = pl.ds((c + NCLS) * BLK, BLK)
                out_ref[b, r0, :] = out_ref[b, r0, :] + po[0:BLK]
                out_ref[b, r1, :] = out_ref[b, r1, :] + po[BLK:2 * BLK]
            return carry

        lax.fori_loop(0, B * H_LOC, att_body, 0)

        right = (me + 1) % N_DEV

        for s in range(N_DEV - 1):
            c_send = (me - s) % N_DEV
            rs_send[s] = out_ref[:, pl.ds(c_send * CH, CH), :].astype(bf16)
            dma = pltpu.make_async_remote_copy(
                src_ref=rs_send.at[s],
                dst_ref=rs_recv.at[s],
                send_sem=rs_send_sems.at[s],
                recv_sem=rs_recv_sems.at[s],
                device_id=(right,),
                device_id_type=pl.DeviceIdType.MESH,
            )
            dma.start()
            dma.wait()
            c_recv = (me - 1 - s) % N_DEV
            r = pl.ds(c_recv * CH, CH)
            out_ref[:, r, :] = out_ref[:, r, :] + rs_recv[s].astype(f32)

        ag_seed[...] = out_ref[:, pl.ds(((me + 1) % N_DEV) * CH, CH),
                               :].astype(bf16)
        for s in range(N_DEV - 1):
            src = ag_seed if s == 0 else ag_recv.at[s - 1]
            dma = pltpu.make_async_remote_copy(
                src_ref=src,
                dst_ref=ag_recv.at[s],
                send_sem=ag_send_sems.at[s],
                recv_sem=ag_recv_sems.at[s],
                device_id=(right,),
                device_id_type=pl.DeviceIdType.MESH,
            )
            dma.start()
            dma.wait()
            c = (me - s) % N_DEV
            out_ref[:, pl.ds(c * CH, CH), :] = ag_recv[s].astype(f32)

    return pl.pallas_call(
        body,
        out_shape=jax.ShapeDtypeStruct((B, SQ, E), jnp.float32),
        in_specs=[
            pl.BlockSpec(memory_space=pltpu.VMEM),
            pl.BlockSpec(memory_space=pltpu.VMEM),
            pl.BlockSpec(memory_space=pltpu.MemorySpace.HBM),
            pl.BlockSpec(memory_space=pltpu.MemorySpace.HBM),
            pl.BlockSpec(memory_space=pltpu.VMEM),
        ],
        out_specs=pl.BlockSpec(memory_space=pltpu.VMEM),
        scratch_shapes=[
            pltpu.VMEM((N_DEV, B, H_LOC, NCLS, 2 * BLK, DH), bf16),
            pltpu.VMEM((N_DEV, B, H_LOC, NCLS, 2 * BLK, DH), bf16),
            pltpu.VMEM((B * H_LOC, SQ, DH), bf16),
            pltpu.VMEM((N_DEV - 1, B, CH, E), bf16),
            pltpu.VMEM((N_DEV - 1, B, CH, E), bf16),
            pltpu.VMEM((B, CH, E), bf16),
            pltpu.VMEM((N_DEV - 1, B, CH, E), bf16),
            pltpu.SemaphoreType.DMA((N_DEV - 1,)),
            pltpu.SemaphoreType.DMA((N_DEV - 1,)),
            pltpu.SemaphoreType.DMA((N_DEV - 1,)),
            pltpu.SemaphoreType.DMA((N_DEV - 1,)),
            pltpu.SemaphoreType.DMA((N_DEV - 1,)),
            pltpu.SemaphoreType.DMA((N_DEV - 1,)),
            pltpu.SemaphoreType.DMA((N_DEV - 1,)),
            pltpu.SemaphoreType.DMA((N_DEV - 1,)),
            pltpu.SemaphoreType.DMA((2,)),
        ],
        compiler_params=pltpu.CompilerParams(
            collective_id=0,
            vmem_limit_bytes=60 * 1024 * 1024,
        ),
    )(x, Wq, Kt, Vt, Wo)


# device time: 454191 ns/iter; 1.3567x vs baseline; 1.0302x over previous
import jax
import jax.numpy as jnp
from jax import lax
from jax.experimental import pallas as pl
from jax.experimental.pallas import tpu as pltpu

N_DEV = 8
B = 2
SQ = 512
SKV = 4096
SKV_LOC = 512
H_LOC = 8
DH = 64
E = 768
BLK = 64
NCLS = 4
KROWS = SKV // NCLS
CH = SQ // N_DEV


def kernel(x, Wq, K_ext, V_ext, Wo):
    bf16 = jnp.bfloat16
    f32 = jnp.float32

    def prep(t):
        t = t.astype(bf16).reshape(B, 2, NCLS, BLK, 64, DH)
        t = jnp.transpose(t, (0, 4, 2, 1, 3, 5))
        return t.reshape(B, 64, NCLS, 2 * BLK, DH)

    Kt = prep(K_ext)
    Vt = prep(V_ext)

    def body(x_ref, wq_ref, kt_ref, vt_ref, wo_ref, out_ref,
             kg, vg, qbuf, bf_send, bf_recv, ag_send, ag_recv,
             k_send_sems, k_recv_sems, v_send_sems, v_recv_sems,
             rs_send_sems, rs_recv_sems, ag_send_sems, ag_recv_sems,
             local_sems):
        me = lax.axis_index("i")

        bsem = pltpu.get_barrier_semaphore()
        for d in range(1, N_DEV):
            pl.semaphore_signal(
                bsem, inc=1,
                device_id=((me + d) % N_DEV,),
                device_id_type=pl.DeviceIdType.MESH,
            )
        pl.semaphore_wait(bsem, N_DEV - 1)

        dmas = []
        for h in range(1, N_DEV):
            dst = (me + h) % N_DEV
            for src_ref, dst_buf, ssems, rsems in (
                (kt_ref, kg, k_send_sems, k_recv_sems),
                (vt_ref, vg, v_send_sems, v_recv_sems),
            ):
                dma = pltpu.make_async_remote_copy(
                    src_ref=src_ref.at[:, pl.ds(dst * H_LOC, H_LOC)],
                    dst_ref=dst_buf.at[me],
                    send_sem=ssems.at[h - 1],
                    recv_sem=rsems.at[h - 1],
                    device_id=(dst,),
                    device_id_type=pl.DeviceIdType.MESH,
                )
                dma.start()
                dmas.append(dma)

        local_dmas = []
        for idx, (src_ref, dst_buf) in enumerate(((kt_ref, kg), (vt_ref, vg))):
            ldma = pltpu.make_async_copy(
                src_ref.at[:, pl.ds(me * H_LOC, H_LOC)],
                dst_buf.at[me],
                local_sems.at[idx],
            )
            ldma.start()
            local_dmas.append(ldma)

        wq = wq_ref[...].astype(bf16)
        for b in range(B):
            q = jnp.dot(x_ref[b].astype(bf16), wq,
                        preferred_element_type=f32) * 0.125
            qs = q.astype(bf16)
            for h in range(H_LOC):
                qbuf[b * H_LOC + h] = qs[:, h * DH:(h + 1) * DH]

        out_ref[...] = jnp.zeros((B, SQ, E), f32)

        for ldma in local_dmas:
            ldma.wait()
        for dma in dmas:
            dma.wait()

        def att_body(i, carry):
            b = i // H_LOC
            h = i % H_LOC
            q_full = qbuf[i]
            wo_h = wo_ref[pl.ds(h * DH, DH), :].astype(bf16)
            for c in range(NCLS):
                q_c = jnp.concatenate(
                    [q_full[c * BLK:(c + 1) * BLK],
                     q_full[(c + NCLS) * BLK:(c + NCLS + 1) * BLK]],
                    axis=0)
                k_c = jnp.concatenate(
                    [kg[s, b, h, c] for s in range(N_DEV)], axis=0)
                v_c = jnp.concatenate(
                    [vg[s, b, h, c] for s in range(N_DEV)], axis=0)
                s_ = lax.dot_general(q_c, k_c, (((1,), (1,)), ((), ())),
                                     preferred_element_type=f32)
                mx = jnp.max(s_, axis=-1, keepdims=True)
                w = jnp.exp(s_ - mx)
                den = jnp.sum(w, axis=-1, keepdims=True)
                wn = (w / den).astype(bf16)
                ctx = jnp.dot(wn, v_c, preferred_element_type=f32)
                po = jnp.dot(ctx.astype(bf16), wo_h,
                             preferred_element_type=f32)
                r0 = pl.ds(c * BLK, BLK)
                r1 = pl.ds((c + NCLS) * BLK, BLK)
                out_ref[b, r0, :] = out_ref[b, r0, :] + po[0:BLK]
                out_ref[b, r1, :] = out_ref[b, r1, :] + po[BLK:2 * BLK]
            return carry

        lax.fori_loop(0, B * H_LOC, att_body, 0)

        seg_off = me * 0
        for k in range(3):
            half = 4 >> k
            partner = me ^ (1 << k)
            bit = (me >> k) & 1
            send_off = seg_off + (1 - bit) * half
            keep_off = seg_off + bit * half
            w = pl.ds(0, half * CH)
            bf_send[k, :, 0:half * CH, :] = out_ref[
                :, pl.ds(send_off * CH, half * CH), :].astype(bf16)
            dma = pltpu.make_async_remote_copy(
                src_ref=bf_send.at[k, :, w, :],
                dst_ref=bf_recv.at[k, :, w, :],
                send_sem=rs_send_sems.at[k],
                recv_sem=rs_recv_sems.at[k],
                device_id=(partner,),
                device_id_type=pl.DeviceIdType.MESH,
            )
            dma.start()
            dma.wait()
            r = pl.ds(keep_off * CH, half * CH)
            out_ref[:, r, :] = out_ref[:, r, :] + \
                bf_recv[k, :, 0:half * CH, :].astype(f32)
            seg_off = keep_off

        for k in (2, 1, 0):
            half = 4 >> k
            partner = me ^ (1 << k)
            bit = (me >> k) & 1
            w = pl.ds(0, half * CH)
            ag_send[k, :, 0:half * CH, :] = out_ref[
                :, pl.ds(seg_off * CH, half * CH), :].astype(bf16)
            dma = pltpu.make_async_remote_copy(
                src_ref=ag_send.at[k, :, w, :],
                dst_ref=ag_recv.at[k, :, w, :],
                send_sem=ag_send_sems.at[k],
                recv_sem=ag_recv_sems.at[k],
                device_id=(partner,),
                device_id_type=pl.DeviceIdType.MESH,
            )
            dma.start()
            dma.wait()
            parent_off = seg_off - bit * half
            partner_off = parent_off + (1 - bit) * half
            out_ref[:, pl.ds(partner_off * CH, half * CH), :] = \
                ag_recv[k, :, 0:half * CH, :].astype(f32)
            seg_off = parent_off

    return pl.pallas_call(
        body,
        out_shape=jax.ShapeDtypeStruct((B, SQ, E), jnp.float32),
        in_specs=[
            pl.BlockSpec(memory_space=pltpu.VMEM),
            pl.BlockSpec(memory_space=pltpu.VMEM),
            pl.BlockSpec(memory_space=pltpu.MemorySpace.HBM),
            pl.BlockSpec(memory_space=pltpu.MemorySpace.HBM),
            pl.BlockSpec(memory_space=pltpu.VMEM),
        ],
        out_specs=pl.BlockSpec(memory_space=pltpu.VMEM),
        scratch_shapes=[
            pltpu.VMEM((N_DEV, B, H_LOC, NCLS, 2 * BLK, DH), bf16),
            pltpu.VMEM((N_DEV, B, H_LOC, NCLS, 2 * BLK, DH), bf16),
            pltpu.VMEM((B * H_LOC, SQ, DH), bf16),
            pltpu.VMEM((3, B, 4 * CH, E), bf16),
            pltpu.VMEM((3, B, 4 * CH, E), bf16),
            pltpu.VMEM((3, B, 4 * CH, E), bf16),
            pltpu.VMEM((3, B, 4 * CH, E), bf16),
            pltpu.SemaphoreType.DMA((N_DEV - 1,)),
            pltpu.SemaphoreType.DMA((N_DEV - 1,)),
            pltpu.SemaphoreType.DMA((N_DEV - 1,)),
            pltpu.SemaphoreType.DMA((N_DEV - 1,)),
            pltpu.SemaphoreType.DMA((3,)),
            pltpu.SemaphoreType.DMA((3,)),
            pltpu.SemaphoreType.DMA((3,)),
            pltpu.SemaphoreType.DMA((3,)),
            pltpu.SemaphoreType.DMA((2,)),
        ],
        compiler_params=pltpu.CompilerParams(
            collective_id=0,
            vmem_limit_bytes=60 * 1024 * 1024,
        ),
    )(x, Wq, Kt, Vt, Wo)
